# Initial kernel scaffold; baseline (speedup 1.0000x reference)
import jax
import jax.numpy as jnp
from jax.experimental import pallas as pl


def kernel(x, edge_attr, edge_index, edge_index2, edge_index3, triangle_1_1_1, triangle_1_1_2, triangle_2_2_1, triangle_2_2_2, triangle_1_2_3, triangle_3_3_1, triangle_2_2_3, triangle_3_3_2, triangle_3_3_3, inverse_edge_1, inverse_edge_2, inverse_edge_3, batch0, num_nodes, atom_emb, bond_emb, W1, b1, W2, b2, Wp1, bp1, Wp2, bp2):
    raise NotImplementedError("write your pallas kernel here")


# speedup vs baseline: 1.0097x; 1.0097x over previous
import functools

import jax
import jax.numpy as jnp
from jax import lax
from jax.experimental import pallas as pl
from jax.experimental.pallas import tpu as pltpu

D = 128


def _mlp_stats_body(a_ref, b_ref, w1_ref, b1_ref, o_ref):
    i = pl.program_id(0)
    x = a_ref[...] + b_ref[...]
    p = jnp.dot(x, w1_ref[...], preferred_element_type=jnp.float32)
    p = p + b1_ref[...]

    @pl.when(i == 0)
    def _():
        o_ref[...] = jnp.zeros_like(o_ref)

    o_ref[0, :] += jnp.sum(p, axis=0)
    o_ref[1, :] += jnp.sum(p * p, axis=0)


def _mlp_apply_body(a_ref, b_ref, w1_ref, b1_ref, st_ref, w2_ref, b2_ref,
                    o_ref):
    x = a_ref[...] + b_ref[...]
    p = jnp.dot(x, w1_ref[...], preferred_element_type=jnp.float32)
    p = p + b1_ref[...]
    h = jnp.maximum((p - st_ref[0:1, :]) * st_ref[1:2, :], 0.0)
    o_ref[...] = jnp.dot(h, w2_ref[...], preferred_element_type=jnp.float32) \
        + b2_ref[...]


@functools.partial(jax.jit, static_argnames=("blk",))
def _mlp(a, b, w1, b1, w2, b2, blk):
    rows = a.shape[0]
    assert rows % blk == 0
    nb = rows // blk

    def row_map(i):
        return (i, 0)

    stats = pl.pallas_call(
        _mlp_stats_body,
        grid=(nb,),
        in_specs=[
            pl.BlockSpec((blk, D), row_map),
            pl.BlockSpec((blk, D), row_map),
            pl.BlockSpec((D, D), lambda i: (0, 0)),
            pl.BlockSpec((D,), lambda i: (0,)),
        ],
        out_specs=pl.BlockSpec((2, D), lambda i: (0, 0)),
        out_shape=jax.ShapeDtypeStruct((2, D), jnp.float32),
    )(a, b, w1, b1)
    mu = stats[0] * (1.0 / rows)
    var = stats[1] * (1.0 / rows) - mu * mu
    inv_sd = 1.0 / (jnp.sqrt(jnp.maximum(var, 0.0)) + 1e-5)
    st = jnp.stack([mu, inv_sd])

    return pl.pallas_call(
        _mlp_apply_body,
        grid=(nb,),
        in_specs=[
            pl.BlockSpec((blk, D), row_map),
            pl.BlockSpec((blk, D), row_map),
            pl.BlockSpec((D, D), lambda i: (0, 0)),
            pl.BlockSpec((D,), lambda i: (0,)),
            pl.BlockSpec((2, D), lambda i: (0, 0)),
            pl.BlockSpec((D, D), lambda i: (0, 0)),
            pl.BlockSpec((D,), lambda i: (0,)),
        ],
        out_specs=pl.BlockSpec((blk, D), row_map),
        out_shape=jax.ShapeDtypeStruct((rows, D), jnp.float32),
    )(a, b, w1, b1, st, w2, b2)


def _head_body(g_ref, wp1_ref, bp1_ref, wp2_ref, bp2_ref, o_ref):
    h = jnp.dot(g_ref[...], wp1_ref[...], preferred_element_type=jnp.float32)
    h = h + bp1_ref[...]
    h = jnp.where(h > 0, h, jnp.exp(jnp.minimum(h, 0.0)) - 1.0)
    o_ref[...] = jnp.dot(h, wp2_ref[...], preferred_element_type=jnp.float32) \
        + bp2_ref[...]


@jax.jit
def _head(g, wp1, bp1, wp2, bp2):
    return pl.pallas_call(
        _head_body,
        out_shape=jax.ShapeDtypeStruct((g.shape[0], wp2.shape[1]), jnp.float32),
    )(g, wp1, bp1, wp2, bp2)


def _encode(table, idx):
    out = jnp.zeros((idx.shape[0], table.shape[2]), dtype=table.dtype)
    for c in range(table.shape[0]):
        out = out + jnp.take(table[c], idx[:, c], axis=0)
    return out


def _tri_msg(ea, eb, tri, nseg):
    m = ea[tri[0]] * eb[tri[1]]
    return jax.ops.segment_sum(m, tri[2], num_segments=nseg)


def kernel(x, edge_attr, edge_index, edge_index2, edge_index3,
           triangle_1_1_1, triangle_1_1_2, triangle_2_2_1, triangle_2_2_2,
           triangle_1_2_3, triangle_3_3_1, triangle_2_2_3, triangle_3_3_2,
           triangle_3_3_3, inverse_edge_1, inverse_edge_2, inverse_edge_3,
           batch0, num_nodes, atom_emb, bond_emb, W1, b1, W2, b2,
           Wp1, bp1, Wp2, bp2):
    ei1, ei2, ei3 = edge_index, edge_index2, edge_index3
    nN = x.shape[0]
    G = 256

    h_atom = _encode(atom_emb, x)
    h_atom = h_atom + jnp.asarray(num_nodes - nN, dtype=h_atom.dtype)
    h0 = h_atom
    e1 = _encode(bond_emb, edge_attr)
    e2 = h_atom[ei2[0]] + h_atom[ei2[1]]
    e3 = h_atom[ei3[0]] + h_atom[ei3[1]]
    nE = e1.shape[0]

    for l in range(W1.shape[0]):
        m0 = jax.ops.segment_sum(e1, ei1[1], num_segments=nN)
        m1 = (_tri_msg(e1, e1, triangle_1_1_1, nE)
              + _tri_msg(e2, e2, triangle_2_2_1, nE)
              + _tri_msg(e3, e3, triangle_3_3_1, nE)
              + h0[ei1[0]] * h0[ei1[1]])
        m2 = (_tri_msg(e1, e1, triangle_1_1_2, nE)
              + _tri_msg(e2, e2, triangle_2_2_2, nE)
              + _tri_msg(e3, e3, triangle_3_3_2, nE)
              + h0[ei2[0]] * h0[ei2[1]])
        m3 = (_tri_msg(e1, e2, triangle_1_2_3, nE)
              + _tri_msg(e2, e2, triangle_2_2_3, nE)
              + _tri_msg(e3, e3, triangle_3_3_3, nE)
              + h0[ei3[0]] * h0[ei3[1]])
        h0 = _mlp(h0, m0, W1[l, 0], b1[l, 0], W2[l, 0], b2[l, 0], blk=1000)
        e1n = _mlp(e1, m1, W1[l, 1], b1[l, 1], W2[l, 1], b2[l, 1], blk=1280)
        e1 = 0.5 * (e1n + e1n[inverse_edge_1])
        e2n = _mlp(e2, m2, W1[l, 2], b1[l, 2], W2[l, 2], b2[l, 2], blk=1280)
        e2 = 0.5 * (e2n + e2n[inverse_edge_2])
        e3n = _mlp(e3, m3, W1[l, 3], b1[l, 3], W2[l, 3], b2[l, 3], blk=1280)
        e3 = 0.5 * (e3n + e3n[inverse_edge_3])

    node = h0
    node = node + jax.ops.segment_sum(e1, ei1[0], num_segments=nN)
    node = node + jax.ops.segment_sum(e2, ei2[0], num_segments=nN)
    node = node + jax.ops.segment_sum(e3, ei3[0], num_segments=nN)
    cnt = jax.ops.segment_sum(jnp.ones((nN, 1), dtype=node.dtype), batch0,
                              num_segments=G)
    g = jax.ops.segment_sum(node, batch0, num_segments=G) / jnp.clip(cnt, 1.0)
    return _head(g, Wp1, bp1, Wp2, bp2)
